# baseline (speedup 1.0000x reference)
import functools

import jax
import jax.numpy as jnp
from jax import lax
from jax.experimental import pallas as pl
from jax.experimental.pallas import tpu as pltpu
from jax.experimental.pallas import tpu_sc as plsc

_N = 10000
_NP = 10112
_E = 320000
_B = 128
_NW = 32
_KB = 8
_TB = 10
_TBG = _TB + 1
_NBUF = 2
_EW = _TB * _KB * _B
_EP = _NW * _EW
_RPS = _NP // 16


def _sc_mesh():
    return plsc.VectorSubcoreMesh(core_axis_name="c", subcore_axis_name="s")


def _make_conv(F):
    @functools.partial(
        pl.kernel,
        mesh=_sc_mesh(),
        compiler_params=pltpu.CompilerParams(use_tc_tiling_on_sc=False),
        out_type=jax.ShapeDtypeStruct((2, _NP, F), jnp.float32),
        scratch_types=[
            pltpu.VMEM((_TBG, _KB * _B), jnp.int32),
            pltpu.VMEM((_TB, _KB * _B), jnp.int32),
            pltpu.VMEM((_NBUF, _KB * _B, F), jnp.float32),
            pltpu.VMEM_SHARED((_NP, F), jnp.float32),
            pltpu.SemaphoreType.DMA((_NBUF,)),
        ],
    )
    def conv(src_hbm, dst_hbm, hs_hbm, zeros_hbm, out_hbm,
             src_v, dst_v, rows_v, acc, gsem):
        cid = lax.axis_index("c")
        sid = lax.axis_index("s")
        wid = cid * 16 + sid
        pltpu.sync_copy(src_hbm.at[wid], src_v)
        pltpu.sync_copy(dst_hbm.at[wid], dst_v)
        r0 = sid * _RPS
        pltpu.sync_copy(zeros_hbm.at[pl.ds(r0, _RPS)], acc.at[pl.ds(r0, _RPS)])
        plsc.subcore_barrier()

        for b in range(_NBUF - 1):
            pltpu.async_copy(hs_hbm.at[src_v.at[b]], rows_v.at[b],
                             gsem.at[b])

        def body(j, carry):
            for b in range(_NBUF):
                blk = j * _NBUF + b
                pltpu.async_copy(
                    hs_hbm.at[src_v.at[blk + _NBUF - 1]],
                    rows_v.at[(b + _NBUF - 1) % _NBUF],
                    gsem.at[(b + _NBUF - 1) % _NBUF])
                pltpu.make_async_copy(hs_hbm.at[src_v.at[blk]],
                                      rows_v.at[b], gsem.at[b]).wait()
                pltpu.sync_copy(rows_v.at[b], acc.at[dst_v.at[blk]],
                                add=True)
            return carry

        lax.fori_loop(0, _TB // _NBUF, body, 0)
        for b in range(_NBUF - 1):
            pltpu.make_async_copy(hs_hbm.at[src_v.at[b]], rows_v.at[b],
                                  gsem.at[b]).wait()
        plsc.subcore_barrier()
        pltpu.sync_copy(acc.at[pl.ds(r0, _RPS)],
                        out_hbm.at[cid, pl.ds(r0, _RPS)])

    return conv


def _make_deg():
    @functools.partial(
        pl.kernel,
        mesh=_sc_mesh(),
        compiler_params=pltpu.CompilerParams(use_tc_tiling_on_sc=False),
        out_type=jax.ShapeDtypeStruct((2, _NP, 16), jnp.float32),
        scratch_types=[
            pltpu.VMEM((_TB, _KB * _B), jnp.int32),
            pltpu.VMEM((_KB * _B, 16), jnp.float32),
            pltpu.VMEM_SHARED((_NP, 16), jnp.float32),
            pltpu.SemaphoreType.DMA,
        ],
    )
    def deg(dst_hbm, ones_hbm, zeros_hbm, out_hbm, dst_v, ones_v, acc, sem):
        cid = lax.axis_index("c")
        sid = lax.axis_index("s")
        wid = cid * 16 + sid
        pltpu.sync_copy(dst_hbm.at[wid], dst_v)
        pltpu.sync_copy(ones_hbm, ones_v)
        r0 = sid * _RPS
        pltpu.sync_copy(zeros_hbm.at[pl.ds(r0, _RPS)], acc.at[pl.ds(r0, _RPS)])
        plsc.subcore_barrier()

        def body(j, carry):
            pltpu.async_copy(ones_v, acc.at[dst_v.at[j]], sem, add=True)
            return carry

        lax.fori_loop(0, _TB, body, 0)

        def drain(j, carry):
            pltpu.make_async_copy(ones_v, acc.at[dst_v.at[j]], sem).wait()
            return carry

        lax.fori_loop(0, _TB, drain, 0)
        plsc.subcore_barrier()
        pltpu.sync_copy(acc.at[pl.ds(r0, _RPS)],
                        out_hbm.at[cid, pl.ds(r0, _RPS)])

    return deg


def _tc_pre(degp, xp, W1):
    def body(degp_ref, x_ref, w_ref, dinv_ref, h1_ref, hs1_ref):
        d = degp_ref[...]
        deg = d[0, :, 0:1] + d[1, :, 0:1] + 1.0
        dinv = lax.rsqrt(deg)
        h1 = jnp.dot(x_ref[...], w_ref[...], preferred_element_type=jnp.float32)
        dinv_ref[...] = dinv
        h1_ref[...] = h1
        hs1_ref[...] = h1 * dinv

    return pl.pallas_call(
        body,
        out_shape=(
            jax.ShapeDtypeStruct((_NP, 1), jnp.float32),
            jax.ShapeDtypeStruct((_NP, 32), jnp.float32),
            jax.ShapeDtypeStruct((_NP, 32), jnp.float32),
        ),
    )(degp, xp, W1)


def _tc_mid(sp, h, dinv, b, W, fout):
    def body(s_ref, h_ref, dinv_ref, b_ref, w_ref, h2_ref, hs2_ref):
        s = s_ref[...]
        dinv = dinv_ref[...]
        agg = dinv * (s[0] + s[1]) + (dinv * dinv) * h_ref[...] + b_ref[...]
        hr = jnp.maximum(agg, 0.0)
        h2 = jnp.dot(hr, w_ref[...], preferred_element_type=jnp.float32)
        h2_ref[...] = h2
        hs2_ref[...] = h2 * dinv

    return pl.pallas_call(
        body,
        out_shape=(
            jax.ShapeDtypeStruct((_NP, fout), jnp.float32),
            jax.ShapeDtypeStruct((_NP, fout), jnp.float32),
        ),
    )(sp, h, dinv, b, W)


def _tc_post(sp, h, dinv, b, lin1_W, lin1_b, lin2_W, lin2_b):
    def body(s_ref, h_ref, dinv_ref, b_ref, w1_ref, b1_ref, w2_ref, b2_ref,
             out_ref):
        s = s_ref[...]
        dinv = dinv_ref[...]
        agg = dinv * (s[0] + s[1]) + (dinv * dinv) * h_ref[...] + b_ref[...]
        t = jnp.dot(agg, w1_ref[...], preferred_element_type=jnp.float32)
        t = jnp.maximum(t + b1_ref[...], 0.0)
        out = jnp.dot(t, w2_ref[...], preferred_element_type=jnp.float32)
        out_ref[...] = out + b2_ref[...]

    return pl.pallas_call(
        body,
        out_shape=jax.ShapeDtypeStruct((_NP, 1), jnp.float32),
    )(sp, h, dinv, b, lin1_W, lin1_b, lin2_W, lin2_b)


def kernel(x, edge_index, W1, b1, W2, b2, W3, b3, lin1_W, lin1_b, lin2_W,
           lin2_b):
    pad = jnp.full((2, _EP - _E), _N, jnp.int32)
    ei = jnp.concatenate([edge_index.astype(jnp.int32), pad], axis=1)
    dst3 = ei[1].reshape(_NW, _TB, _KB * _B)
    src3 = jnp.concatenate(
        [ei[0].reshape(_NW, _TB, _KB * _B),
         jnp.zeros((_NW, _TBG - _TB, _KB * _B), jnp.int32)], axis=1)
    xp = jnp.pad(x, ((0, _NP - _N), (0, 0)))
    z16 = jnp.zeros((_NP, 16), jnp.float32)
    z32 = jnp.zeros((_NP, 32), jnp.float32)
    ones = jnp.ones((_KB * _B, 16), jnp.float32)

    degp = _make_deg()(dst3, ones, z16)
    dinv, h1, hs1 = _tc_pre(degp, xp, W1)
    s1 = _make_conv(32)(src3, dst3, hs1, z32)
    h2, hs2 = _tc_mid(s1, h1, dinv, b1.reshape(1, 32), W2, 16)
    s2 = _make_conv(16)(src3, dst3, hs2, z16)
    h3, hs3 = _tc_mid(s2, h2, dinv, b2.reshape(1, 16), W3, 16)
    s3 = _make_conv(16)(src3, dst3, hs3, z16)
    out = _tc_post(s3, h3, dinv, b3.reshape(1, 16), lin1_W,
                   lin1_b.reshape(1, 8), lin2_W, lin2_b.reshape(1, 1))
    return out[:_N]


# speedup vs baseline: 3.0921x; 3.0921x over previous
import functools

import jax
import jax.numpy as jnp
from jax import lax
from jax.experimental import pallas as pl
from jax.experimental.pallas import tpu as pltpu
from jax.experimental.pallas import tpu_sc as plsc

_N = 10000
_NP = 10112
_E = 320000
_B = 128
_NW = 32
_NBLK = 80
_EW = _NBLK * _B
_EP = _NW * _EW
_RPS = _NP // 16


def _sc_mesh():
    return plsc.VectorSubcoreMesh(core_axis_name="c", subcore_axis_name="s")


def _make_conv(F):
    @functools.partial(
        pl.kernel,
        mesh=_sc_mesh(),
        compiler_params=pltpu.CompilerParams(use_tc_tiling_on_sc=False),
        out_type=jax.ShapeDtypeStruct((2, _NP, F), jnp.float32),
        scratch_types=[
            pltpu.VMEM((_NBLK, _B), jnp.int32),
            pltpu.VMEM((_NBLK, _B), jnp.int32),
            pltpu.VMEM((_B, F), jnp.float32),
            pltpu.VMEM_SHARED((_NP, F), jnp.float32),
            pltpu.VMEM_SHARED((_NP, F), jnp.float32),
            pltpu.SemaphoreType.DMA,
        ],
    )
    def conv(src_hbm, dst_hbm, hs_hbm, zeros_hbm, out_hbm,
             src_v, dst_v, rows_v, tab, acc, gsem):
        cid = lax.axis_index("c")
        sid = lax.axis_index("s")
        wid = cid * 16 + sid
        pltpu.sync_copy(src_hbm.at[wid], src_v)
        pltpu.sync_copy(dst_hbm.at[wid], dst_v)
        r0 = sid * _RPS
        pltpu.sync_copy(hs_hbm.at[pl.ds(r0, _RPS)], tab.at[pl.ds(r0, _RPS)])
        pltpu.sync_copy(zeros_hbm.at[pl.ds(r0, _RPS)], acc.at[pl.ds(r0, _RPS)])
        plsc.subcore_barrier()

        def body(j, carry):
            pltpu.async_copy(tab.at[src_v.at[j]], rows_v, gsem).wait()
            pltpu.sync_copy(rows_v, acc.at[dst_v.at[j]], add=True)
            return carry

        lax.fori_loop(0, _NBLK, body, 0)
        plsc.subcore_barrier()
        pltpu.sync_copy(acc.at[pl.ds(r0, _RPS)],
                        out_hbm.at[cid, pl.ds(r0, _RPS)])

    return conv


def _make_deg():
    @functools.partial(
        pl.kernel,
        mesh=_sc_mesh(),
        compiler_params=pltpu.CompilerParams(use_tc_tiling_on_sc=False),
        out_type=jax.ShapeDtypeStruct((2, _NP, 16), jnp.float32),
        scratch_types=[
            pltpu.VMEM((_NBLK, _B), jnp.int32),
            pltpu.VMEM((_B, 16), jnp.float32),
            pltpu.VMEM_SHARED((_NP, 16), jnp.float32),
            pltpu.SemaphoreType.DMA,
        ],
    )
    def deg(dst_hbm, ones_hbm, zeros_hbm, out_hbm, dst_v, ones_v, acc, sem):
        cid = lax.axis_index("c")
        sid = lax.axis_index("s")
        wid = cid * 16 + sid
        pltpu.sync_copy(dst_hbm.at[wid], dst_v)
        pltpu.sync_copy(ones_hbm, ones_v)
        r0 = sid * _RPS
        pltpu.sync_copy(zeros_hbm.at[pl.ds(r0, _RPS)], acc.at[pl.ds(r0, _RPS)])
        plsc.subcore_barrier()

        def body(j, carry):
            pltpu.async_copy(ones_v, acc.at[dst_v.at[j]], sem, add=True)
            return carry

        lax.fori_loop(0, _NBLK, body, 0)

        def drain(j, carry):
            pltpu.make_async_copy(ones_v, acc.at[dst_v.at[j]], sem).wait()
            return carry

        lax.fori_loop(0, _NBLK, drain, 0)
        plsc.subcore_barrier()
        pltpu.sync_copy(acc.at[pl.ds(r0, _RPS)],
                        out_hbm.at[cid, pl.ds(r0, _RPS)])

    return deg


def _tc_pre(degp, xp, W1):
    def body(degp_ref, x_ref, w_ref, dinv_ref, h1_ref, hs1_ref):
        d = degp_ref[...]
        deg = d[0, :, 0:1] + d[1, :, 0:1] + 1.0
        dinv = lax.rsqrt(deg)
        h1 = jnp.dot(x_ref[...], w_ref[...], preferred_element_type=jnp.float32)
        dinv_ref[...] = dinv
        h1_ref[...] = h1
        hs1_ref[...] = h1 * dinv

    return pl.pallas_call(
        body,
        out_shape=(
            jax.ShapeDtypeStruct((_NP, 1), jnp.float32),
            jax.ShapeDtypeStruct((_NP, 32), jnp.float32),
            jax.ShapeDtypeStruct((_NP, 32), jnp.float32),
        ),
    )(degp, xp, W1)


def _tc_mid(sp, h, dinv, b, W, fout):
    def body(s_ref, h_ref, dinv_ref, b_ref, w_ref, h2_ref, hs2_ref):
        s = s_ref[...]
        dinv = dinv_ref[...]
        agg = dinv * (s[0] + s[1]) + (dinv * dinv) * h_ref[...] + b_ref[...]
        hr = jnp.maximum(agg, 0.0)
        h2 = jnp.dot(hr, w_ref[...], preferred_element_type=jnp.float32)
        h2_ref[...] = h2
        hs2_ref[...] = h2 * dinv

    return pl.pallas_call(
        body,
        out_shape=(
            jax.ShapeDtypeStruct((_NP, fout), jnp.float32),
            jax.ShapeDtypeStruct((_NP, fout), jnp.float32),
        ),
    )(sp, h, dinv, b, W)


def _tc_post(sp, h, dinv, b, lin1_W, lin1_b, lin2_W, lin2_b):
    def body(s_ref, h_ref, dinv_ref, b_ref, w1_ref, b1_ref, w2_ref, b2_ref,
             out_ref):
        s = s_ref[...]
        dinv = dinv_ref[...]
        agg = dinv * (s[0] + s[1]) + (dinv * dinv) * h_ref[...] + b_ref[...]
        t = jnp.dot(agg, w1_ref[...], preferred_element_type=jnp.float32)
        t = jnp.maximum(t + b1_ref[...], 0.0)
        out = jnp.dot(t, w2_ref[...], preferred_element_type=jnp.float32)
        out_ref[...] = out + b2_ref[...]

    return pl.pallas_call(
        body,
        out_shape=jax.ShapeDtypeStruct((_NP, 1), jnp.float32),
    )(sp, h, dinv, b, lin1_W, lin1_b, lin2_W, lin2_b)


def kernel(x, edge_index, W1, b1, W2, b2, W3, b3, lin1_W, lin1_b, lin2_W,
           lin2_b):
    pad = jnp.full((2, _EP - _E), _N, jnp.int32)
    ei = jnp.concatenate([edge_index.astype(jnp.int32), pad], axis=1)
    src3 = ei[0].reshape(_NW, _NBLK, _B)
    dst3 = ei[1].reshape(_NW, _NBLK, _B)
    xp = jnp.pad(x, ((0, _NP - _N), (0, 0)))
    z16 = jnp.zeros((_NP, 16), jnp.float32)
    z32 = jnp.zeros((_NP, 32), jnp.float32)
    ones = jnp.ones((_B, 16), jnp.float32)

    degp = _make_deg()(dst3, ones, z16)
    dinv, h1, hs1 = _tc_pre(degp, xp, W1)
    s1 = _make_conv(32)(src3, dst3, hs1, z32)
    h2, hs2 = _tc_mid(s1, h1, dinv, b1.reshape(1, 32), W2, 16)
    s2 = _make_conv(16)(src3, dst3, hs2, z16)
    h3, hs3 = _tc_mid(s2, h2, dinv, b2.reshape(1, 16), W3, 16)
    s3 = _make_conv(16)(src3, dst3, hs3, z16)
    out = _tc_post(s3, h3, dinv, b3.reshape(1, 16), lin1_W,
                   lin1_b.reshape(1, 8), lin2_W, lin2_b.reshape(1, 1))
    return out[:_N]


# speedup vs baseline: 3.4562x; 1.1177x over previous
import functools

import jax
import jax.numpy as jnp
from jax import lax
from jax.experimental import pallas as pl
from jax.experimental.pallas import tpu as pltpu
from jax.experimental.pallas import tpu_sc as plsc

_N = 10000
_NP = 10112
_E = 320000
_B = 128
_NW = 32
_NBLK = 80
_EW = _NBLK * _B
_EP = _NW * _EW
_RPS = _NP // 16


def _sc_mesh():
    return plsc.VectorSubcoreMesh(core_axis_name="c", subcore_axis_name="s")


def _make_conv(F):
    @functools.partial(
        pl.kernel,
        mesh=_sc_mesh(),
        compiler_params=pltpu.CompilerParams(use_tc_tiling_on_sc=False),
        out_type=jax.ShapeDtypeStruct((2, _NP, F), jnp.float32),
        scratch_types=[
            pltpu.VMEM((_NBLK, _B), jnp.int32),
            pltpu.VMEM((_NBLK, _B), jnp.int32),
            pltpu.VMEM((2, _B, F), jnp.float32),
            pltpu.VMEM_SHARED((_NP, F), jnp.float32),
            pltpu.VMEM_SHARED((_NP, F), jnp.float32),
            pltpu.SemaphoreType.DMA((2,)),
        ],
    )
    def conv(src_hbm, dst_hbm, hs_hbm, zeros_hbm, out_hbm,
             src_v, dst_v, rows_v, tab, acc, gsem):
        cid = lax.axis_index("c")
        sid = lax.axis_index("s")
        wid = cid * 16 + sid
        pltpu.sync_copy(src_hbm.at[wid], src_v)
        pltpu.sync_copy(dst_hbm.at[wid], dst_v)
        r0 = sid * _RPS
        pltpu.sync_copy(hs_hbm.at[pl.ds(r0, _RPS)], tab.at[pl.ds(r0, _RPS)])
        pltpu.sync_copy(zeros_hbm.at[pl.ds(r0, _RPS)], acc.at[pl.ds(r0, _RPS)])
        plsc.subcore_barrier()

        pltpu.async_copy(tab.at[src_v.at[0]], rows_v.at[0], gsem.at[0])

        def body(j, carry):
            for b in range(2):
                blk = j * 2 + b
                nxt = blk + 1
                pltpu.async_copy(tab.at[src_v.at[jnp.minimum(nxt, _NBLK - 1)]],
                                 rows_v.at[1 - b], gsem.at[1 - b])
                pltpu.make_async_copy(tab.at[src_v.at[blk]], rows_v.at[b],
                                      gsem.at[b]).wait()
                pltpu.sync_copy(rows_v.at[b], acc.at[dst_v.at[blk]], add=True)
            return carry

        lax.fori_loop(0, _NBLK // 2, body, 0)
        pltpu.make_async_copy(tab.at[src_v.at[0]], rows_v.at[0],
                              gsem.at[0]).wait()
        plsc.subcore_barrier()
        pltpu.sync_copy(acc.at[pl.ds(r0, _RPS)],
                        out_hbm.at[cid, pl.ds(r0, _RPS)])

    return conv


def _make_deg():
    @functools.partial(
        pl.kernel,
        mesh=_sc_mesh(),
        compiler_params=pltpu.CompilerParams(use_tc_tiling_on_sc=False),
        out_type=jax.ShapeDtypeStruct((2, _NP, 16), jnp.float32),
        scratch_types=[
            pltpu.VMEM((_NBLK, _B), jnp.int32),
            pltpu.VMEM((_B, 16), jnp.float32),
            pltpu.VMEM_SHARED((_NP, 16), jnp.float32),
            pltpu.SemaphoreType.DMA,
        ],
    )
    def deg(dst_hbm, ones_hbm, zeros_hbm, out_hbm, dst_v, ones_v, acc, sem):
        cid = lax.axis_index("c")
        sid = lax.axis_index("s")
        wid = cid * 16 + sid
        pltpu.sync_copy(dst_hbm.at[wid], dst_v)
        pltpu.sync_copy(ones_hbm, ones_v)
        r0 = sid * _RPS
        pltpu.sync_copy(zeros_hbm.at[pl.ds(r0, _RPS)], acc.at[pl.ds(r0, _RPS)])
        plsc.subcore_barrier()

        def body(j, carry):
            pltpu.async_copy(ones_v, acc.at[dst_v.at[j]], sem, add=True)
            return carry

        lax.fori_loop(0, _NBLK, body, 0)

        def drain(j, carry):
            pltpu.make_async_copy(ones_v, acc.at[dst_v.at[j]], sem).wait()
            return carry

        lax.fori_loop(0, _NBLK, drain, 0)
        plsc.subcore_barrier()
        pltpu.sync_copy(acc.at[pl.ds(r0, _RPS)],
                        out_hbm.at[cid, pl.ds(r0, _RPS)])

    return deg


def _tc_pre(degp, xp, W1):
    def body(degp_ref, x_ref, w_ref, dinv_ref, h1_ref, hs1_ref):
        d = degp_ref[...]
        deg = d[0, :, 0:1] + d[1, :, 0:1] + 1.0
        dinv = lax.rsqrt(deg)
        h1 = jnp.dot(x_ref[...], w_ref[...], preferred_element_type=jnp.float32)
        dinv_ref[...] = dinv
        h1_ref[...] = h1
        hs1_ref[...] = h1 * dinv

    return pl.pallas_call(
        body,
        out_shape=(
            jax.ShapeDtypeStruct((_NP, 1), jnp.float32),
            jax.ShapeDtypeStruct((_NP, 32), jnp.float32),
            jax.ShapeDtypeStruct((_NP, 32), jnp.float32),
        ),
    )(degp, xp, W1)


def _tc_mid(sp, h, dinv, b, W, fout):
    def body(s_ref, h_ref, dinv_ref, b_ref, w_ref, h2_ref, hs2_ref):
        s = s_ref[...]
        dinv = dinv_ref[...]
        agg = dinv * (s[0] + s[1]) + (dinv * dinv) * h_ref[...] + b_ref[...]
        hr = jnp.maximum(agg, 0.0)
        h2 = jnp.dot(hr, w_ref[...], preferred_element_type=jnp.float32)
        h2_ref[...] = h2
        hs2_ref[...] = h2 * dinv

    return pl.pallas_call(
        body,
        out_shape=(
            jax.ShapeDtypeStruct((_NP, fout), jnp.float32),
            jax.ShapeDtypeStruct((_NP, fout), jnp.float32),
        ),
    )(sp, h, dinv, b, W)


def _tc_post(sp, h, dinv, b, lin1_W, lin1_b, lin2_W, lin2_b):
    def body(s_ref, h_ref, dinv_ref, b_ref, w1_ref, b1_ref, w2_ref, b2_ref,
             out_ref):
        s = s_ref[...]
        dinv = dinv_ref[...]
        agg = dinv * (s[0] + s[1]) + (dinv * dinv) * h_ref[...] + b_ref[...]
        t = jnp.dot(agg, w1_ref[...], preferred_element_type=jnp.float32)
        t = jnp.maximum(t + b1_ref[...], 0.0)
        out = jnp.dot(t, w2_ref[...], preferred_element_type=jnp.float32)
        out_ref[...] = out + b2_ref[...]

    return pl.pallas_call(
        body,
        out_shape=jax.ShapeDtypeStruct((_NP, 1), jnp.float32),
    )(sp, h, dinv, b, lin1_W, lin1_b, lin2_W, lin2_b)


def kernel(x, edge_index, W1, b1, W2, b2, W3, b3, lin1_W, lin1_b, lin2_W,
           lin2_b):
    pad = jnp.full((2, _EP - _E), _N, jnp.int32)
    ei = jnp.concatenate([edge_index.astype(jnp.int32), pad], axis=1)
    src3 = ei[0].reshape(_NW, _NBLK, _B)
    dst3 = ei[1].reshape(_NW, _NBLK, _B)
    xp = jnp.pad(x, ((0, _NP - _N), (0, 0)))
    z16 = jnp.zeros((_NP, 16), jnp.float32)
    z32 = jnp.zeros((_NP, 32), jnp.float32)
    ones = jnp.ones((_B, 16), jnp.float32)

    degp = _make_deg()(dst3, ones, z16)
    dinv, h1, hs1 = _tc_pre(degp, xp, W1)
    s1 = _make_conv(32)(src3, dst3, hs1, z32)
    h2, hs2 = _tc_mid(s1, h1, dinv, b1.reshape(1, 32), W2, 16)
    s2 = _make_conv(16)(src3, dst3, hs2, z16)
    h3, hs3 = _tc_mid(s2, h2, dinv, b2.reshape(1, 16), W3, 16)
    s3 = _make_conv(16)(src3, dst3, hs3, z16)
    out = _tc_post(s3, h3, dinv, b3.reshape(1, 16), lin1_W,
                   lin1_b.reshape(1, 8), lin2_W, lin2_b.reshape(1, 1))
    return out[:_N]


# speedup vs baseline: 4.1437x; 1.1989x over previous
import functools

import jax
import jax.numpy as jnp
from jax import lax
from jax.experimental import pallas as pl
from jax.experimental.pallas import tpu as pltpu
from jax.experimental.pallas import tpu_sc as plsc

_N = 10000
_NP = 10112
_NPG = _NP // 4
_E = 320000
_B = 128
_NW = 32
_NBLK = 80
_EW = _NBLK * _B
_EP = _NW * _EW
_RPS = _NP // 16
_F = 32


def _sc_mesh():
    return plsc.VectorSubcoreMesh(core_axis_name="c", subcore_axis_name="s")


def _make_conv():
    @functools.partial(
        pl.kernel,
        mesh=_sc_mesh(),
        compiler_params=pltpu.CompilerParams(use_tc_tiling_on_sc=False),
        out_type=jax.ShapeDtypeStruct((2, _NP, _F), jnp.float32),
        scratch_types=[
            pltpu.VMEM((_NBLK, _B), jnp.int32),
            pltpu.VMEM((_NBLK, _B), jnp.int32),
            pltpu.VMEM((2, _B, _F), jnp.float32),
            pltpu.VMEM_SHARED((_NP, _F), jnp.float32),
            pltpu.VMEM_SHARED((_NP, _F), jnp.float32),
            pltpu.SemaphoreType.DMA((2,)),
        ],
    )
    def conv(src_hbm, dst_hbm, hs_hbm, zeros_hbm, out_hbm,
             src_v, dst_v, rows_v, tab, acc, gsem):
        cid = lax.axis_index("c")
        sid = lax.axis_index("s")
        wid = cid * 16 + sid
        pltpu.sync_copy(src_hbm.at[wid], src_v)
        pltpu.sync_copy(dst_hbm.at[wid], dst_v)
        r0 = sid * _RPS
        pltpu.sync_copy(hs_hbm.at[pl.ds(r0, _RPS)], tab.at[pl.ds(r0, _RPS)])
        pltpu.sync_copy(zeros_hbm.at[pl.ds(r0, _RPS)], acc.at[pl.ds(r0, _RPS)])
        plsc.subcore_barrier()

        pltpu.async_copy(tab.at[src_v.at[0]], rows_v.at[0], gsem.at[0])

        def body(j, carry):
            for b in range(2):
                blk = j * 2 + b
                nxt = blk + 1
                pltpu.async_copy(tab.at[src_v.at[jnp.minimum(nxt, _NBLK - 1)]],
                                 rows_v.at[1 - b], gsem.at[1 - b])
                pltpu.make_async_copy(tab.at[src_v.at[blk]], rows_v.at[b],
                                      gsem.at[b]).wait()
                pltpu.sync_copy(rows_v.at[b], acc.at[dst_v.at[blk]], add=True)
            return carry

        lax.fori_loop(0, _NBLK // 2, body, 0)
        pltpu.make_async_copy(tab.at[src_v.at[0]], rows_v.at[0],
                              gsem.at[0]).wait()
        plsc.subcore_barrier()
        pltpu.sync_copy(acc.at[pl.ds(r0, _RPS)],
                        out_hbm.at[cid, pl.ds(r0, _RPS)])

    return conv


def _make_deg():
    @functools.partial(
        pl.kernel,
        mesh=_sc_mesh(),
        compiler_params=pltpu.CompilerParams(use_tc_tiling_on_sc=False),
        out_type=jax.ShapeDtypeStruct((2, _NP, _F), jnp.float32),
        scratch_types=[
            pltpu.VMEM((_NBLK, _B), jnp.int32),
            pltpu.VMEM((_B, _F), jnp.float32),
            pltpu.VMEM_SHARED((_NP, _F), jnp.float32),
            pltpu.SemaphoreType.DMA,
        ],
    )
    def deg(dst_hbm, ones_hbm, zeros_hbm, out_hbm, dst_v, ones_v, acc, sem):
        cid = lax.axis_index("c")
        sid = lax.axis_index("s")
        wid = cid * 16 + sid
        pltpu.sync_copy(dst_hbm.at[wid], dst_v)
        pltpu.sync_copy(ones_hbm, ones_v)
        r0 = sid * _RPS
        pltpu.sync_copy(zeros_hbm.at[pl.ds(r0, _RPS)], acc.at[pl.ds(r0, _RPS)])
        plsc.subcore_barrier()

        def body(j, carry):
            pltpu.async_copy(ones_v, acc.at[dst_v.at[j]], sem, add=True)
            return carry

        lax.fori_loop(0, _NBLK, body, 0)

        def drain(j, carry):
            pltpu.make_async_copy(ones_v, acc.at[dst_v.at[j]], sem).wait()
            return carry

        lax.fori_loop(0, _NBLK, drain, 0)
        plsc.subcore_barrier()
        pltpu.sync_copy(acc.at[pl.ds(r0, _RPS)],
                        out_hbm.at[cid, pl.ds(r0, _RPS)])

    return deg


def _tc_pre(degp, x4, W1k):
    def body(degp_ref, x_ref, w_ref, dinv_ref, h1_ref, hs1_ref):
        d = degp_ref[...]
        dinv = lax.rsqrt(d[0] + d[1] + 1.0)
        h1 = jnp.dot(x_ref[...], w_ref[...], preferred_element_type=jnp.float32)
        dinv_ref[...] = dinv
        h1_ref[...] = h1
        hs1_ref[...] = h1 * dinv

    return pl.pallas_call(
        body,
        out_shape=(
            jax.ShapeDtypeStruct((_NPG, 128), jnp.float32),
            jax.ShapeDtypeStruct((_NPG, 128), jnp.float32),
            jax.ShapeDtypeStruct((_NPG, 128), jnp.float32),
        ),
    )(degp, x4, W1k)


def _tc_mid(sp, h, dinv, bt, Wk):
    def body(s_ref, h_ref, dinv_ref, b_ref, w_ref, h2_ref, hs2_ref):
        s = s_ref[...]
        dinv = dinv_ref[...]
        agg = dinv * (s[0] + s[1]) + (dinv * dinv) * h_ref[...] + b_ref[...]
        hr = jnp.maximum(agg, 0.0)
        h2 = jnp.dot(hr, w_ref[...], preferred_element_type=jnp.float32)
        h2_ref[...] = h2
        hs2_ref[...] = h2 * dinv

    return pl.pallas_call(
        body,
        out_shape=(
            jax.ShapeDtypeStruct((_NPG, 128), jnp.float32),
            jax.ShapeDtypeStruct((_NPG, 128), jnp.float32),
        ),
    )(sp, h, dinv, bt, Wk)


def _tc_post(sp, h, dinv, bt, L1k, l1bt, L2k, l2bt):
    def body(s_ref, h_ref, dinv_ref, b_ref, w1_ref, b1_ref, w2_ref, b2_ref,
             out_ref):
        s = s_ref[...]
        dinv = dinv_ref[...]
        agg = dinv * (s[0] + s[1]) + (dinv * dinv) * h_ref[...] + b_ref[...]
        t = jnp.dot(agg, w1_ref[...], preferred_element_type=jnp.float32)
        t = jnp.maximum(t + b1_ref[...], 0.0)
        out = jnp.dot(t, w2_ref[...], preferred_element_type=jnp.float32)
        out_ref[...] = out + b2_ref[...]

    return pl.pallas_call(
        body,
        out_shape=jax.ShapeDtypeStruct((_NPG, 4), jnp.float32),
    )(sp, h, dinv, bt, L1k, l1bt, L2k, l2bt)


def kernel(x, edge_index, W1, b1, W2, b2, W3, b3, lin1_W, lin1_b, lin2_W,
           lin2_b):
    f32 = jnp.float32
    pad = jnp.full((2, _EP - _E), _N, jnp.int32)
    ei = jnp.concatenate([edge_index.astype(jnp.int32), pad], axis=1)
    src3 = ei[0].reshape(_NW, _NBLK, _B)
    dst3 = ei[1].reshape(_NW, _NBLK, _B)
    xp = jnp.pad(x, ((0, _NP - _N), (0, 0)))
    x4 = xp.reshape(_NPG, 512)
    z32 = jnp.zeros((_NP, _F), f32)
    ones = jnp.ones((_B, _F), f32)
    eye4 = jnp.eye(4, dtype=f32)

    W1k = jnp.kron(eye4, W1)
    W2k = jnp.kron(eye4, jnp.pad(W2, ((0, 0), (0, 16))))
    W3k = jnp.kron(eye4, jnp.pad(W3, ((0, 16), (0, 16))))
    L1k = jnp.kron(eye4, jnp.pad(lin1_W, ((0, 16), (0, 0))))
    L2k = jnp.kron(eye4, lin2_W)
    b1t = jnp.tile(b1, 4)[None]
    b2t = jnp.tile(jnp.pad(b2, (0, 16)), 4)[None]
    b3t = jnp.tile(jnp.pad(b3, (0, 16)), 4)[None]
    l1bt = jnp.tile(lin1_b, 4)[None]
    l2bt = jnp.tile(lin2_b, 4)[None]

    conv = _make_conv()
    degp = _make_deg()(dst3, ones, z32)
    dinv, h1, hs1 = _tc_pre(degp.reshape(2, _NPG, 128), x4, W1k)
    s1 = conv(src3, dst3, hs1.reshape(_NP, _F), z32)
    h2, hs2 = _tc_mid(s1.reshape(2, _NPG, 128), h1, dinv, b1t, W2k)
    s2 = conv(src3, dst3, hs2.reshape(_NP, _F), z32)
    h3, hs3 = _tc_mid(s2.reshape(2, _NPG, 128), h2, dinv, b2t, W3k)
    s3 = conv(src3, dst3, hs3.reshape(_NP, _F), z32)
    out4 = _tc_post(s3.reshape(2, _NPG, 128), h3, dinv, b3t,
                    L1k, l1bt, L2k, l2bt)
    return out4.reshape(_NP, 1)[:_N]
